# baseline (speedup 1.0000x reference)
import dataclasses

import jax
import jax.numpy as jnp
from jax.experimental import pallas as pl
from jax.experimental.pallas import tpu as pltpu
from jax.experimental.pallas import tpu_sc as plsc

T = 4
N = 116
HID = 16
BERT = 768
E = 2048
D_CAT = BERT + HID
NEG = -1e30
NP = 128
SEG = (N * NP) // 8

_SC_PARAMS = pltpu.CompilerParams()
if "needs_layout_passes" in pltpu.CompilerParams.__dataclass_fields__:
    _SC_PARAMS = dataclasses.replace(_SC_PARAMS, needs_layout_passes=False)


def _hist_body(hg_ref, out_ref, vbuf, ebuf, hist_e, sem1, sem2):
    wid = jax.lax.axis_index("c") * 16 + jax.lax.axis_index("s")
    t = wid // 8
    w = wid % 8
    lo = w * SEG

    cp1 = pltpu.async_copy(hg_ref.at[t, 0], vbuf, sem1)
    cp2 = pltpu.async_copy(hg_ref.at[t, 1], ebuf, sem2)

    zeros16 = jnp.zeros((16,), jnp.float32)

    @pl.loop(0, SEG, step=16, unroll=8)
    def _(i):
        hist_e[pl.ds(i, 16)] = zeros16

    cp1.wait()
    cp2.wait()

    ones16 = jnp.full((16,), 1.0, jnp.float32)

    @pl.loop(0, E, step=16, unroll=4)
    def _(j):
        v16 = vbuf[pl.ds(j, 16)]
        e16 = ebuf[pl.ds(j, 16)]
        idx = e16 * NP + v16 - lo
        mask = plsc.bitcast(idx, jnp.uint32) < SEG
        plsc.addupdate_scatter(hist_e, [idx], ones16, mask=mask)

    pltpu.async_copy(hist_e, out_ref.at[wid], sem1).wait()


def _histograms(hgs):
    k = pl.kernel(
        _hist_body,
        out_type=jax.ShapeDtypeStruct((32, SEG), jnp.float32),
        mesh=plsc.VectorSubcoreMesh(core_axis_name="c", subcore_axis_name="s"),
        scratch_types=[
            pltpu.VMEM((E,), jnp.int32),
            pltpu.VMEM((E,), jnp.int32),
            pltpu.VMEM((SEG,), jnp.float32),
            pltpu.SemaphoreType.DMA,
            pltpu.SemaphoreType.DMA,
        ],
        compiler_params=_SC_PARAMS,
    )
    return k(hgs)


def _lstm_gates_t(z, c):
    i = jax.nn.sigmoid(z[0:HID])
    f = jax.nn.sigmoid(z[HID:2 * HID])
    g = jnp.tanh(z[2 * HID:3 * HID])
    o = jax.nn.sigmoid(z[3 * HID:4 * HID])
    c = f * c + i * g
    return o * jnp.tanh(c), c


_WEC = 0
_WIH2 = 784
_PR = 848
_WECS1 = 856
_WECS2 = 864
_WHH1 = 872
_WHH2 = 936
_WIH1 = 1000
_B1 = 1008
_B2 = 1072
_WQIN = 1136
_WOUT = 1152
_WFC = 1168
_BFC = 1176
_BEC = 1184
_WVC = 1192
_ROWS = 1200


def _tc_body(ch_ref, ne_ref, pk, out_ref):
    f32 = jnp.float32
    mm = (((1,), (0,)), ((), ()))
    cdims = (((1,), (1,)), ((), ()))
    oo = (((0,), (0,)), ((), ()))
    crow = (((0,), (1,)), ((), ()))

    def dot(a, b, d):
        return jax.lax.dot_general(a, b, d, preferred_element_type=f32,
                                   precision=jax.lax.Precision.HIGHEST)

    wih1r = pk[_WIH1:_WIH1 + 1, 0:4 * HID]
    whh1 = pk[_WHH1:_WHH1 + 4 * HID, 0:HID]
    whh2 = pk[_WHH2:_WHH2 + 4 * HID, 0:HID]
    b1c = pk[_B1:_B1 + 4 * HID, 0:1]
    b2c = pk[_B2:_B2 + 4 * HID, 0:1]
    wvcr = pk[_WVC:_WVC + 1, 0:HID]
    wih2 = pk[_WIH2:_WIH2 + 4 * HID, :]

    h = jnp.zeros((HID, N), f32)
    c = jnp.zeros((HID, N), f32)
    new_prices = []
    sv_cols = []
    for t in range(T):
        pr_row = pk[_PR + t:_PR + t + 1, 0:N]
        z = (dot(wih1r, pr_row, oo)
             + dot(whh1, h, mm)
             + b1c)
        h, c = _lstm_gates_t(z, c)
        new_prices.append(h)
        sv_cols.append(dot(h, wvcr, crow))

    wct1 = dot(wih2, pk[_WEC:_WEC + HID, :], cdims)
    wct2 = dot(wih2, pk[_WEC + HID:_WEC + D_CAT, :], cdims)
    bc = dot(wih2, pk[_BEC:_BEC + 1, :], cdims)
    wecs1 = pk[_WECS1:_WECS1 + 1, 0:HID]
    wecs2 = pk[_WECS2:_WECS2 + 1, 0:BERT]

    zin = []
    for t in range(T):
        cev = ch_ref[t][:, 0:N]
        cve = cev.T
        pe = new_prices[t]
        sv_col = sv_cols[t]
        mk1 = cve > 0
        m1 = jnp.max(jnp.where(mk1, sv_col, NEG), axis=0, keepdims=True)
        m1 = jnp.where(m1 > 0.5 * NEG, m1, 0.0)
        a1 = jnp.where(mk1, cve * jnp.exp(sv_col - m1), 0.0)
        den1 = jnp.sum(a1, axis=0, keepdims=True)
        he = dot(pe, a1, mm) / (den1 + 1e-9)

        ae = ne_ref[t]
        sc_col = (dot(he, wecs1, crow)
                  + dot(ae, wecs2, cdims))
        mk2 = cev > 0
        m2 = jnp.max(jnp.where(mk2, sc_col, NEG), axis=0, keepdims=True)
        m2 = jnp.where(m2 > 0.5 * NEG, m2, 0.0)
        a2 = jnp.where(mk2, cev * jnp.exp(sc_col - m2), 0.0)
        den2 = jnp.sum(a2, axis=0, keepdims=True)
        hcw = (dot(wct1, he, mm)
               + dot(wct2, ae, cdims))
        aggw = dot(hcw, a2, mm) / (den2 + 1e-9)
        zin.append(jnp.where(den2 > 0, aggw + bc, 0.0))

    h2 = jnp.zeros((HID, N), f32)
    c2 = jnp.zeros((HID, N), f32)
    la = []
    for t in range(T):
        z = zin[t] + dot(whh2, h2, mm) + b2c
        h2, c2 = _lstm_gates_t(z, c2)
        la.append(h2 + new_prices[t])

    q = la[T - 1]
    qp = dot(pk[_WQIN:_WQIN + HID, 0:HID], q, mm)
    scores = [jnp.sum(qp * la[t], axis=0, keepdims=True) for t in range(T)]
    m = scores[0]
    for t in range(1, T):
        m = jnp.maximum(m, scores[t])
    ws = [jnp.exp(scores[t] - m) for t in range(T)]
    den = ws[0]
    for t in range(1, T):
        den = den + ws[t]
    mix = ws[0] * la[0]
    for t in range(1, T):
        mix = mix + ws[t] * la[t]
    mix = mix / den
    mq = jnp.concatenate([mix, q], axis=0)
    comb = jnp.tanh(dot(pk[_WOUT:_WOUT + HID, 0:2 * HID], mq, mm))
    out_ref[...] = (dot(pk[_WFC:_WFC + 2, 0:HID], comb, mm)
                    + pk[_BFC:_BFC + 2, 0:1]).T


def kernel(hgs, node_embs, prices, Wih1, Whh1, b1, w_vc, w_ec_score, W_ec,
           b_ec, Wih2, Whh2, b2, W_qin, W_out, W_fc, b_fc):
    f32 = jnp.float32
    ch = _histograms(hgs.astype(jnp.int32)).reshape(T, N, NP)

    def pad(x, rows):
        r, c = x.shape
        return jnp.pad(x, ((0, rows - r), (0, D_CAT - c)))

    packed = jnp.concatenate([
        W_ec,
        Wih2,
        pad(prices.reshape(T, N), 8),
        pad(w_ec_score[:HID].reshape(1, HID), 8),
        pad(w_ec_score[HID:].reshape(1, BERT), 8),
        pad(Whh1, 64),
        pad(Whh2, 64),
        pad(Wih1.reshape(1, 4 * HID), 8),
        pad(b1.reshape(4 * HID, 1), 64),
        pad(b2.reshape(4 * HID, 1), 64),
        pad(W_qin, 16),
        pad(W_out, 16),
        pad(W_fc, 8),
        pad(b_fc.reshape(2, 1), 8),
        pad(b_ec.reshape(1, D_CAT), 8),
        pad(w_vc.reshape(1, HID), 8),
    ], axis=0).astype(f32)

    return pl.pallas_call(
        _tc_body,
        out_shape=jax.ShapeDtypeStruct((N, 2), f32),
    )(ch, node_embs, packed)


# speedup vs baseline: 1.1712x; 1.1712x over previous
import dataclasses

import jax
import jax.numpy as jnp
from jax.experimental import pallas as pl
from jax.experimental.pallas import tpu as pltpu
from jax.experimental.pallas import tpu_sc as plsc

T = 4
N = 116
HID = 16
BERT = 768
E = 2048
D_CAT = BERT + HID
NEG = -1e30
NP = 128
SEG = (N * NP) // 8

_SC_PARAMS = pltpu.CompilerParams()
if "needs_layout_passes" in pltpu.CompilerParams.__dataclass_fields__:
    _SC_PARAMS = dataclasses.replace(_SC_PARAMS, needs_layout_passes=False)


def _hist_body(hg_ref, out_ref, vbuf, ebuf, hist_e, sem1, sem2):
    wid = jax.lax.axis_index("c") * 16 + jax.lax.axis_index("s")
    t = wid // 8
    w = wid % 8
    lo = w * SEG

    cp1 = pltpu.async_copy(hg_ref.at[t, 0], vbuf, sem1)
    cp2 = pltpu.async_copy(hg_ref.at[t, 1], ebuf, sem2)

    zeros16 = jnp.zeros((16,), jnp.float32)

    @pl.loop(0, SEG, step=16, unroll=8)
    def _(i):
        hist_e[pl.ds(i, 16)] = zeros16

    cp1.wait()
    cp2.wait()

    ones16 = jnp.full((16,), 1.0, jnp.float32)

    @pl.loop(0, E, step=16, unroll=4)
    def _(j):
        v16 = vbuf[pl.ds(j, 16)]
        e16 = ebuf[pl.ds(j, 16)]
        idx = e16 * NP + v16 - lo
        mask = plsc.bitcast(idx, jnp.uint32) < SEG
        plsc.addupdate_scatter(hist_e, [idx], ones16, mask=mask)

    pltpu.async_copy(hist_e, out_ref.at[wid], sem1).wait()


def _histograms(hgs):
    k = pl.kernel(
        _hist_body,
        out_type=jax.ShapeDtypeStruct((32, SEG), jnp.float32),
        mesh=plsc.VectorSubcoreMesh(core_axis_name="c", subcore_axis_name="s"),
        scratch_types=[
            pltpu.VMEM((E,), jnp.int32),
            pltpu.VMEM((E,), jnp.int32),
            pltpu.VMEM((SEG,), jnp.float32),
            pltpu.SemaphoreType.DMA,
            pltpu.SemaphoreType.DMA,
        ],
        compiler_params=_SC_PARAMS,
    )
    return k(hgs)


def _lstm_gates_t(z, c):
    i = jax.nn.sigmoid(z[0:HID])
    f = jax.nn.sigmoid(z[HID:2 * HID])
    g = jnp.tanh(z[2 * HID:3 * HID])
    o = jax.nn.sigmoid(z[3 * HID:4 * HID])
    c = f * c + i * g
    return o * jnp.tanh(c), c


_WEC = 0
_WIH2 = 784
_PR = 848
_WECS1 = 856
_WECS2 = 864
_WHH1 = 872
_WHH2 = 936
_WIH1 = 1000
_B1 = 1008
_B2 = 1072
_WQIN = 1136
_WOUT = 1152
_WFC = 1168
_BFC = 1176
_BEC = 1184
_WVC = 1192
_ROWS = 1200


def _tc_body(ch_ref, ne_ref, pk, out_ref):
    f32 = jnp.float32
    mm = (((1,), (0,)), ((), ()))
    cdims = (((1,), (1,)), ((), ()))
    oo = (((0,), (0,)), ((), ()))
    crow = (((0,), (1,)), ((), ()))

    def dot(a, b, d):
        return jax.lax.dot_general(a, b, d, preferred_element_type=f32)

    wih1r = pk[_WIH1:_WIH1 + 1, 0:4 * HID]
    whh1 = pk[_WHH1:_WHH1 + 4 * HID, 0:HID]
    whh2 = pk[_WHH2:_WHH2 + 4 * HID, 0:HID]
    b1c = pk[_B1:_B1 + 4 * HID, 0:1]
    b2c = pk[_B2:_B2 + 4 * HID, 0:1]
    wvcr = pk[_WVC:_WVC + 1, 0:HID]
    wih2 = pk[_WIH2:_WIH2 + 4 * HID, :]

    h = jnp.zeros((HID, N), f32)
    c = jnp.zeros((HID, N), f32)
    new_prices = []
    sv_cols = []
    for t in range(T):
        pr_row = pk[_PR + t:_PR + t + 1, 0:N]
        z = (dot(wih1r, pr_row, oo)
             + dot(whh1, h, mm)
             + b1c)
        h, c = _lstm_gates_t(z, c)
        new_prices.append(h)
        sv_cols.append(dot(h, wvcr, crow))

    wct1 = dot(wih2, pk[_WEC:_WEC + HID, :], cdims)
    wct2 = dot(wih2, pk[_WEC + HID:_WEC + D_CAT, :], cdims)
    bc = dot(wih2, pk[_BEC:_BEC + 1, :], cdims)
    wecs1 = pk[_WECS1:_WECS1 + 1, 0:HID]
    wecs2 = pk[_WECS2:_WECS2 + 1, 0:BERT]

    zin = []
    for t in range(T):
        cev = ch_ref[t][:, 0:N]
        cve = cev.T
        pe = new_prices[t]
        sv_col = sv_cols[t]
        mk1 = cve > 0
        m1 = jnp.max(jnp.where(mk1, sv_col, NEG), axis=0, keepdims=True)
        m1 = jnp.where(m1 > 0.5 * NEG, m1, 0.0)
        a1 = jnp.where(mk1, cve * jnp.exp(sv_col - m1), 0.0)
        den1 = jnp.sum(a1, axis=0, keepdims=True)
        he = dot(pe, a1, mm) / (den1 + 1e-9)

        ae = ne_ref[t]
        sc_col = (dot(he, wecs1, crow)
                  + dot(ae, wecs2, cdims))
        mk2 = cev > 0
        m2 = jnp.max(jnp.where(mk2, sc_col, NEG), axis=0, keepdims=True)
        m2 = jnp.where(m2 > 0.5 * NEG, m2, 0.0)
        a2 = jnp.where(mk2, cev * jnp.exp(sc_col - m2), 0.0)
        den2 = jnp.sum(a2, axis=0, keepdims=True)
        hcw = (dot(wct1, he, mm)
               + dot(wct2, ae, cdims))
        aggw = dot(hcw, a2, mm) / (den2 + 1e-9)
        zin.append(jnp.where(den2 > 0, aggw + bc, 0.0))

    h2 = jnp.zeros((HID, N), f32)
    c2 = jnp.zeros((HID, N), f32)
    la = []
    for t in range(T):
        z = zin[t] + dot(whh2, h2, mm) + b2c
        h2, c2 = _lstm_gates_t(z, c2)
        la.append(h2 + new_prices[t])

    q = la[T - 1]
    qp = dot(pk[_WQIN:_WQIN + HID, 0:HID], q, mm)
    scores = [jnp.sum(qp * la[t], axis=0, keepdims=True) for t in range(T)]
    m = scores[0]
    for t in range(1, T):
        m = jnp.maximum(m, scores[t])
    ws = [jnp.exp(scores[t] - m) for t in range(T)]
    den = ws[0]
    for t in range(1, T):
        den = den + ws[t]
    mix = ws[0] * la[0]
    for t in range(1, T):
        mix = mix + ws[t] * la[t]
    mix = mix / den
    mq = jnp.concatenate([mix, q], axis=0)
    comb = jnp.tanh(dot(pk[_WOUT:_WOUT + HID, 0:2 * HID], mq, mm))
    out_ref[...] = (dot(pk[_WFC:_WFC + 2, 0:HID], comb, mm)
                    + pk[_BFC:_BFC + 2, 0:1]).T


def kernel(hgs, node_embs, prices, Wih1, Whh1, b1, w_vc, w_ec_score, W_ec,
           b_ec, Wih2, Whh2, b2, W_qin, W_out, W_fc, b_fc):
    f32 = jnp.float32
    ch = _histograms(hgs.astype(jnp.int32)).reshape(T, N, NP)

    def pad(x, rows):
        r, c = x.shape
        return jnp.pad(x, ((0, rows - r), (0, D_CAT - c)))

    packed = jnp.concatenate([
        W_ec,
        Wih2,
        pad(prices.reshape(T, N), 8),
        pad(w_ec_score[:HID].reshape(1, HID), 8),
        pad(w_ec_score[HID:].reshape(1, BERT), 8),
        pad(Whh1, 64),
        pad(Whh2, 64),
        pad(Wih1.reshape(1, 4 * HID), 8),
        pad(b1.reshape(4 * HID, 1), 64),
        pad(b2.reshape(4 * HID, 1), 64),
        pad(W_qin, 16),
        pad(W_out, 16),
        pad(W_fc, 8),
        pad(b_fc.reshape(2, 1), 8),
        pad(b_ec.reshape(1, D_CAT), 8),
        pad(w_vc.reshape(1, HID), 8),
    ], axis=0).astype(f32)

    return pl.pallas_call(
        _tc_body,
        out_shape=jax.ShapeDtypeStruct((N, 2), f32),
    )(ch, node_embs, packed)
